# baseline (speedup 1.0000x reference)
import functools

import jax
import jax.numpy as jnp
from jax import lax
from jax.experimental import pallas as pl
from jax.experimental.pallas import tpu as pltpu
from jax.experimental.pallas import tpu_sc as plsc

N_NODES = 50000
N_EDGES = 1600000
NS = 16
L = 16

N_PAD = 50176
TSPAN = N_PAD // NS
EPT = N_EDGES // NS
EBLK = 10000
NBLK = EPT // EBLK
NPAIR = NBLK // 2
LAST_W = N_NODES - (N_PAD - TSPAN)
SU = 5


def _mask_body(edge_hbm, vparam_hbm, out_hbm, pub_hbm,
               reach, colA, colB, rowbuf, vparam, redbuf, outbuf, flagbuf,
               allflags, dirty, shared_flags, semA, semB):
    sid = lax.axis_index("s")
    ebase = sid * EPT

    def start_col(b, cbuf, sem):
        off = ebase + b * EBLK
        pltpu.make_async_copy(edge_hbm.at[pl.ds(N_EDGES + off, EBLK)], cbuf,
                              sem).start()

    def wait_col(cbuf, sem):
        pltpu.make_async_copy(edge_hbm.at[pl.ds(0, EBLK)], cbuf,
                              sem).wait()

    start_col(0, colA, semA)
    start_col(1, colB, semB)

    pltpu.sync_copy(vparam_hbm, vparam)
    vtx = vparam[...]

    zero_f = jnp.zeros((L,), jnp.float32)
    one_f = jnp.ones((L,), jnp.float32)
    ninf = jnp.full((L,), -jnp.inf, jnp.float32)
    zero_i = jnp.zeros((L,), jnp.int32)
    one_i = jnp.ones((L,), jnp.int32)

    dirty[0] = 0

    def scan_blk(b, cbuf):
        @plsc.parallel_loop(0, EBLK, step=L, unroll=SU, carry=zero_i)
        def acc(i, a):
            return a + jnp.where(cbuf[pl.ds(i, L)] == vtx, one_i, zero_i)
        cnt = jnp.max(acc)

        @pl.when(cnt > 0)
        def _():
            off = ebase + b * EBLK
            pltpu.sync_copy(edge_hbm.at[pl.ds(off, EBLK)], rowbuf)

            @pl.when(dirty[0] == 0)
            def _():
                @plsc.parallel_loop(0, N_PAD, step=L, unroll=8)
                def _z(i):
                    reach[pl.ds(i, L)] = zero_f
            dirty[0] = 1

            @plsc.parallel_loop(0, EBLK, step=L, unroll=SU)
            def _s2(i):
                s = pl.ds(i, L)
                cv = cbuf[s]
                rv = rowbuf[s]
                hit = (cv == vtx) & (rv != vtx)
                plsc.store_scatter(reach, [rv], one_f, mask=hit)

    def pair(p, c):
        wait_col(colA, semA)
        scan_blk(2 * p, colA)

        @pl.when(p < NPAIR - 1)
        def _():
            start_col(2 * p + 2, colA, semA)

        wait_col(colB, semB)
        scan_blk(2 * p + 1, colB)

        @pl.when(p < NPAIR - 1)
        def _():
            start_col(2 * p + 3, colB, semB)
        return c
    lax.fori_loop(0, NPAIR, pair, 0)

    d = dirty[0]

    @pl.when(d > 0)
    def _():
        pltpu.sync_copy(reach, pub_hbm.at[pl.ds(sid * N_PAD, N_PAD)])

    flagbuf[pl.ds(0, L)] = jnp.full((L,), d, dtype=jnp.int32)
    pltpu.sync_copy(flagbuf, shared_flags.at[pl.ds(sid * L, L)])
    plsc.subcore_barrier()

    myoff = sid * TSPAN

    @plsc.parallel_loop(0, TSPAN, step=L, unroll=8)
    def _zo(i):
        outbuf[pl.ds(i, L)] = zero_f

    pltpu.sync_copy(shared_flags, allflags)

    for t in range(NS):
        ft = jnp.max(allflags[pl.ds(t * L, L)])

        @pl.when(ft > 0)
        def _(_t=t):
            pltpu.sync_copy(pub_hbm.at[pl.ds(_t * N_PAD + myoff, TSPAN)],
                            redbuf)

            @plsc.parallel_loop(0, TSPAN, step=L, unroll=8)
            def _ab(i):
                s0 = pl.ds(i, L)
                outbuf[s0] = outbuf[s0] + redbuf[s0]

    neg1 = vtx == jnp.full((L,), -1, dtype=jnp.int32)

    @plsc.parallel_loop(0, TSPAN, step=L, unroll=8)
    def _fv(i):
        s0 = pl.ds(i, L)
        a = outbuf[s0]
        o = jnp.where(a > zero_f, zero_f, ninf)
        o = jnp.where(neg1, zero_f, o)
        outbuf[s0] = o

    is_last = sid == NS - 1

    @pl.when(jnp.logical_not(is_last))
    def _():
        pltpu.sync_copy(outbuf, out_hbm.at[pl.ds(myoff, TSPAN)])

    @pl.when(is_last)
    def _():
        pltpu.sync_copy(outbuf.at[pl.ds(0, LAST_W)],
                        out_hbm.at[pl.ds(myoff, LAST_W)])


_sc_mask = functools.partial(
    pl.kernel,
    mesh=plsc.VectorSubcoreMesh(core_axis_name="c", subcore_axis_name="s",
                                num_cores=1),
    out_type=(jax.ShapeDtypeStruct((N_NODES,), jnp.float32),
              jax.ShapeDtypeStruct((NS * N_PAD,), jnp.float32)),
    compiler_params=pltpu.CompilerParams(needs_layout_passes=False),
    scratch_types=[
        pltpu.VMEM((N_PAD,), jnp.float32),
        pltpu.VMEM((EBLK,), jnp.int32),
        pltpu.VMEM((EBLK,), jnp.int32),
        pltpu.VMEM((EBLK,), jnp.int32),
        pltpu.VMEM((L,), jnp.int32),
        pltpu.VMEM((TSPAN,), jnp.float32),
        pltpu.VMEM((TSPAN,), jnp.float32),
        pltpu.VMEM((L,), jnp.int32),
        pltpu.VMEM((NS * L,), jnp.int32),
        pltpu.SMEM((1,), jnp.int32),
        pltpu.VMEM_SHARED((NS * L,), jnp.int32),
        pltpu.SemaphoreType.DMA,
        pltpu.SemaphoreType.DMA,
    ],
)(_mask_body)


def kernel(logits, edge_index, vertex):
    del logits
    vparam = jnp.full((L,), vertex, dtype=jnp.int32)
    mask, _ = _sc_mask(edge_index.reshape(-1), vparam)
    return mask.reshape(-1, 1)


# speedup vs baseline: 1.0037x; 1.0037x over previous
import functools

import jax
import jax.numpy as jnp
from jax import lax
from jax.experimental import pallas as pl
from jax.experimental.pallas import tpu as pltpu
from jax.experimental.pallas import tpu_sc as plsc

N_NODES = 50000
N_EDGES = 1600000
NS = 16
L = 16

N_PAD = 50176
TSPAN = N_PAD // NS
EPT = N_EDGES // NS
EBLK = 10000
NBLK = EPT // EBLK
NPAIR = NBLK // 2
LAST_W = N_NODES - (N_PAD - TSPAN)
SU = 5


NRING = 3


def _mask_body(edge_hbm, vparam_hbm, out_hbm, pub_hbm,
               reach, colA, colB, colC, rowbuf, vparam, redbuf, outbuf,
               flagbuf, allflags, dirty, shared_flags, semA, semB, semC):
    sid = lax.axis_index("s")
    ebase = sid * EPT
    cbufs = (colA, colB, colC)
    sems = (semA, semB, semC)

    def start_col(b, cbuf, sem):
        off = ebase + b * EBLK
        pltpu.make_async_copy(edge_hbm.at[pl.ds(N_EDGES + off, EBLK)], cbuf,
                              sem).start()

    def wait_col(cbuf, sem):
        pltpu.make_async_copy(edge_hbm.at[pl.ds(0, EBLK)], cbuf,
                              sem).wait()

    for b in range(NRING):
        start_col(b, cbufs[b], sems[b])

    pltpu.sync_copy(vparam_hbm, vparam)
    vtx = vparam[...]

    zero_f = jnp.zeros((L,), jnp.float32)
    one_f = jnp.ones((L,), jnp.float32)
    ninf = jnp.full((L,), -jnp.inf, jnp.float32)
    zero_i = jnp.zeros((L,), jnp.int32)
    one_i = jnp.ones((L,), jnp.int32)

    dirty[0] = 0

    def scan_blk(b, cbuf):
        @plsc.parallel_loop(0, EBLK, step=L, unroll=SU, carry=zero_i)
        def acc(i, a):
            return a + jnp.where(cbuf[pl.ds(i, L)] == vtx, one_i, zero_i)
        cnt = jnp.max(acc)

        @pl.when(cnt > 0)
        def _():
            off = ebase + b * EBLK
            pltpu.sync_copy(edge_hbm.at[pl.ds(off, EBLK)], rowbuf)

            @pl.when(dirty[0] == 0)
            def _():
                @plsc.parallel_loop(0, N_PAD, step=L, unroll=8)
                def _z(i):
                    reach[pl.ds(i, L)] = zero_f
            dirty[0] = 1

            @plsc.parallel_loop(0, EBLK, step=L, unroll=SU)
            def _s2(i):
                s = pl.ds(i, L)
                cv = cbuf[s]
                rv = rowbuf[s]
                hit = (cv == vtx) & (rv != vtx)
                plsc.store_scatter(reach, [rv], one_f, mask=hit)

    for b in range(NBLK):
        s = b % NRING
        wait_col(cbufs[s], sems[s])
        scan_blk(b, cbufs[s])
        if b + NRING < NBLK:
            start_col(b + NRING, cbufs[s], sems[s])

    d = dirty[0]

    @pl.when(d > 0)
    def _():
        pltpu.sync_copy(reach, pub_hbm.at[pl.ds(sid * N_PAD, N_PAD)])

    flagbuf[pl.ds(0, L)] = jnp.full((L,), d, dtype=jnp.int32)
    pltpu.sync_copy(flagbuf, shared_flags.at[pl.ds(sid * L, L)])
    plsc.subcore_barrier()

    myoff = sid * TSPAN

    @plsc.parallel_loop(0, TSPAN, step=L, unroll=8)
    def _zo(i):
        outbuf[pl.ds(i, L)] = zero_f

    pltpu.sync_copy(shared_flags, allflags)

    for t in range(NS):
        ft = jnp.max(allflags[pl.ds(t * L, L)])

        @pl.when(ft > 0)
        def _(_t=t):
            pltpu.sync_copy(pub_hbm.at[pl.ds(_t * N_PAD + myoff, TSPAN)],
                            redbuf)

            @plsc.parallel_loop(0, TSPAN, step=L, unroll=8)
            def _ab(i):
                s0 = pl.ds(i, L)
                outbuf[s0] = outbuf[s0] + redbuf[s0]

    neg1 = vtx == jnp.full((L,), -1, dtype=jnp.int32)

    @plsc.parallel_loop(0, TSPAN, step=L, unroll=8)
    def _fv(i):
        s0 = pl.ds(i, L)
        a = outbuf[s0]
        o = jnp.where(a > zero_f, zero_f, ninf)
        o = jnp.where(neg1, zero_f, o)
        outbuf[s0] = o

    is_last = sid == NS - 1

    @pl.when(jnp.logical_not(is_last))
    def _():
        pltpu.sync_copy(outbuf, out_hbm.at[pl.ds(myoff, TSPAN)])

    @pl.when(is_last)
    def _():
        pltpu.sync_copy(outbuf.at[pl.ds(0, LAST_W)],
                        out_hbm.at[pl.ds(myoff, LAST_W)])


_sc_mask = functools.partial(
    pl.kernel,
    mesh=plsc.VectorSubcoreMesh(core_axis_name="c", subcore_axis_name="s",
                                num_cores=1),
    out_type=(jax.ShapeDtypeStruct((N_NODES,), jnp.float32),
              jax.ShapeDtypeStruct((NS * N_PAD,), jnp.float32)),
    compiler_params=pltpu.CompilerParams(needs_layout_passes=False),
    scratch_types=[
        pltpu.VMEM((N_PAD,), jnp.float32),
        pltpu.VMEM((EBLK,), jnp.int32),
        pltpu.VMEM((EBLK,), jnp.int32),
        pltpu.VMEM((EBLK,), jnp.int32),
        pltpu.VMEM((EBLK,), jnp.int32),
        pltpu.VMEM((L,), jnp.int32),
        pltpu.VMEM((TSPAN,), jnp.float32),
        pltpu.VMEM((TSPAN,), jnp.float32),
        pltpu.VMEM((L,), jnp.int32),
        pltpu.VMEM((NS * L,), jnp.int32),
        pltpu.SMEM((1,), jnp.int32),
        pltpu.VMEM_SHARED((NS * L,), jnp.int32),
        pltpu.SemaphoreType.DMA,
        pltpu.SemaphoreType.DMA,
        pltpu.SemaphoreType.DMA,
    ],
)(_mask_body)


def kernel(logits, edge_index, vertex):
    del logits
    vparam = jnp.full((L,), vertex, dtype=jnp.int32)
    mask, _ = _sc_mask(edge_index.reshape(-1), vparam)
    return mask.reshape(-1, 1)
